# Initial kernel scaffold; baseline (speedup 1.0000x reference)
import jax
import jax.numpy as jnp
from jax.experimental import pallas as pl


def kernel(input, weight):
    raise NotImplementedError("write your pallas kernel here")


# speedup vs baseline: 1.1666x; 1.1666x over previous
import functools

import jax
import jax.numpy as jnp
from jax import lax
from jax.experimental import pallas as pl
from jax.experimental.pallas import tpu as pltpu
from jax.experimental.pallas import tpu_sc as plsc

D = 128
CH = 128
NC = 2
NS = 16
NW = NC * NS


@functools.cache
def _make_gather(B):
    BPW = B // NW
    NCHUNK = BPW // CH

    mesh = plsc.VectorSubcoreMesh(core_axis_name="c", subcore_axis_name="s")

    @functools.partial(
        pl.kernel,
        out_type=jax.ShapeDtypeStruct((B, D), jnp.float32),
        mesh=mesh,
        scratch_types=[
            pltpu.VMEM((BPW,), jnp.int32),
            pltpu.VMEM((2, CH, D), jnp.float32),
            pltpu.SemaphoreType.DMA,
        ],
    )
    def gather_kernel(idx_hbm, table_hbm, out_hbm, idx_v, rows_v, sem):
        wid = lax.axis_index("s") * NC + lax.axis_index("c")
        base = wid * BPW
        pltpu.sync_copy(idx_hbm.at[pl.ds(base, BPW)], idx_v)

        @pl.loop(0, NCHUNK)
        def chunk(c):
            pltpu.async_copy(
                table_hbm.at[idx_v.at[pl.ds(c * CH, CH)]], rows_v.at[0], sem
            ).wait()
            pltpu.sync_copy(rows_v.at[0], out_hbm.at[pl.ds(base + c * CH, CH)])

    return gather_kernel


def kernel(input, weight):
    B, L = input.shape
    n = B * L
    idx = input.astype(jnp.int32).reshape(n)
    out = _make_gather(n)(idx, weight)
    return out.reshape(B, L, D)


# speedup vs baseline: 1.2997x; 1.1141x over previous
import functools

import jax
import jax.numpy as jnp
from jax import lax
from jax.experimental import pallas as pl
from jax.experimental.pallas import tpu as pltpu
from jax.experimental.pallas import tpu_sc as plsc

D = 128
CH = 128
NC = 2
NS = 16
NW = NC * NS
NBUF = 6
LOOKAHEAD = NBUF - 2


@functools.cache
def _make_gather(B):
    BPW = B // NW
    NCHUNK = BPW // CH

    mesh = plsc.VectorSubcoreMesh(core_axis_name="c", subcore_axis_name="s")

    @functools.partial(
        pl.kernel,
        out_type=jax.ShapeDtypeStruct((B, D), jnp.float32),
        mesh=mesh,
        scratch_types=[
            pltpu.VMEM((BPW,), jnp.int32),
            pltpu.VMEM((NBUF, CH, D), jnp.float32),
            pltpu.SemaphoreType.DMA,
            pltpu.SemaphoreType.DMA,
        ],
    )
    def gather_kernel(idx_hbm, table_hbm, out_hbm, idx_v, rows_v, gsem, osem):
        wid = lax.axis_index("s") * NC + lax.axis_index("c")
        base = wid * BPW
        pltpu.sync_copy(idx_hbm.at[pl.ds(base, BPW)], idx_v)

        def g_start(c):
            b = lax.rem(c, NBUF) if not isinstance(c, int) else c % NBUF
            pltpu.async_copy(
                table_hbm.at[idx_v.at[pl.ds(c * CH, CH)]], rows_v.at[b], gsem
            )

        def g_wait():
            pltpu.make_async_copy(
                table_hbm.at[pl.ds(0, CH)], rows_v.at[0], gsem
            ).wait()

        def o_start(c):
            b = lax.rem(c, NBUF) if not isinstance(c, int) else c % NBUF
            pltpu.async_copy(
                rows_v.at[b], out_hbm.at[pl.ds(base + c * CH, CH)], osem
            )

        def o_wait():
            pltpu.make_async_copy(
                rows_v.at[0], out_hbm.at[pl.ds(base, CH)], osem
            ).wait()

        for c in range(LOOKAHEAD):
            g_start(c)

        @pl.loop(0, NCHUNK + 2)
        def body(c):
            @pl.when(c >= 2)
            def _():
                o_wait()

            @pl.when(c + LOOKAHEAD < NCHUNK)
            def _():
                g_start(c + LOOKAHEAD)

            @pl.when(c < NCHUNK)
            def _():
                g_wait()
                o_start(c)

    return gather_kernel


def kernel(input, weight):
    B, L = input.shape
    n = B * L
    idx = input.astype(jnp.int32).reshape(n)
    out = _make_gather(n)(idx, weight)
    return out.reshape(B, L, D)
